# baseline (speedup 1.0000x reference)
import functools

import jax
import jax.numpy as jnp
from jax import lax
from jax.experimental import pallas as pl
from jax.experimental.pallas import tpu as pltpu
from jax.experimental.pallas import tpu_sc as plsc

N = 10000
E = 320000
D_IN = 128
D_HID = 128
N_CLS = 4

NC = 2
NS = 16
L = 16
NW = NC * NS

CH = 128
GRP = 80
E_PAD = NW * GRP * CH
NPAD = 10240
RPW = NPAD // NS
FR = NPAD * N_CLS // NS
PAD_IDX = N


def _worker(c, s):
    return s * NC + c


def _sc_degree(ei128):
    mesh = plsc.VectorSubcoreMesh(core_axis_name="c", subcore_axis_name="s")

    @functools.partial(
        pl.kernel,
        mesh=mesh,
        out_type=jax.ShapeDtypeStruct((NC, NPAD), jnp.float32),
        scratch_types=[
            pltpu.VMEM((GRP, CH), jnp.int32),
            pltpu.VMEM((CH,), jnp.float32),
            pltpu.VMEM((RPW,), jnp.float32),
            pltpu.VMEM_SHARED((NPAD,), jnp.float32),
            pltpu.SemaphoreType.DMA,
        ],
    )
    def k(ei_hbm, out_hbm, idx_v, ones_v, buf_v, acc_sh, sem):
        c = lax.axis_index("c")
        s = lax.axis_index("s")
        w = _worker(c, s)
        for i in range(CH // L):
            ones_v[pl.ds(i * L, L)] = jnp.ones((L,), jnp.float32)

        def zero_body(i, carry):
            buf_v[pl.ds(i * L, L)] = jnp.zeros((L,), jnp.float32)
            return carry

        lax.fori_loop(0, RPW // L, zero_body, 0)
        sl = pl.ds(s * RPW, RPW)
        pltpu.sync_copy(buf_v, acc_sh.at[sl])
        pltpu.async_copy(
            ei_hbm.at[1].at[pl.ds(w * GRP, GRP)], idx_v, sem).wait()
        plsc.subcore_barrier()

        def body(j, carry):
            pltpu.sync_copy(ones_v, acc_sh.at[idx_v.at[j]], add=True)
            return carry

        lax.fori_loop(0, GRP, body, 0)
        plsc.subcore_barrier()
        pltpu.sync_copy(acc_sh.at[sl], out_hbm.at[c].at[sl])

    return k(ei128)


def _sc_agg(ei128, y, zeros, d):
    mesh = plsc.VectorSubcoreMesh(core_axis_name="c", subcore_axis_name="s")

    @functools.partial(
        pl.kernel,
        mesh=mesh,
        out_type=jax.ShapeDtypeStruct((NC, NPAD, d), jnp.float32),
        scratch_types=[
            pltpu.VMEM((GRP // 2, CH), jnp.int32),
            pltpu.VMEM((GRP // 2, CH), jnp.int32),
            pltpu.VMEM((CH, d), jnp.float32),
            pltpu.VMEM((CH, d), jnp.float32),
            pltpu.VMEM_SHARED((NPAD, d), jnp.float32),
            pltpu.SemaphoreType.DMA,
            pltpu.SemaphoreType.DMA,
        ],
    )
    def k(ei_hbm, y_hbm, z_hbm, out_hbm, si_v, di_v, rows0_v,
          rows1_v, acc_sh, sem0, sem1):
        c = lax.axis_index("c")
        s = lax.axis_index("s")
        w = _worker(c, s)
        HGRP = GRP // 2
        sl = pl.ds(s * RPW, RPW)
        pltpu.async_copy(z_hbm.at[sl], acc_sh.at[sl], sem0)
        pltpu.make_async_copy(z_hbm.at[sl], acc_sh.at[sl], sem0).wait()
        plsc.subcore_barrier()

        def run_half(h):
            base = w * GRP + h * HGRP
            pltpu.async_copy(
                ei_hbm.at[0].at[pl.ds(base, HGRP)], si_v, sem1).wait()
            pltpu.async_copy(
                ei_hbm.at[1].at[pl.ds(base, HGRP)], di_v, sem1).wait()
            pltpu.async_copy(y_hbm.at[si_v.at[0]], rows0_v, sem0)

            def body(jj, carry):
                j0 = 2 * jj
                j1 = j0 + 1
                pltpu.async_copy(y_hbm.at[si_v.at[j1]], rows1_v, sem1)
                pltpu.make_async_copy(
                    y_hbm.at[si_v.at[j0]], rows0_v, sem0).wait()
                pltpu.sync_copy(rows0_v, acc_sh.at[di_v.at[j0]], add=True)

                @pl.when(jj + 1 < HGRP // 2)
                def _():
                    pltpu.async_copy(y_hbm.at[si_v.at[j0 + 2]], rows0_v, sem0)

                pltpu.make_async_copy(
                    y_hbm.at[si_v.at[j1]], rows1_v, sem1).wait()
                pltpu.sync_copy(rows1_v, acc_sh.at[di_v.at[j1]], add=True)
                return carry

            lax.fori_loop(0, HGRP // 2, body, 0)

        run_half(0)
        run_half(1)
        plsc.subcore_barrier()
        pltpu.sync_copy(acc_sh.at[sl], out_hbm.at[c].at[sl, :])

    return k(ei128, y, zeros)


def _sc_agg_flat(ei128, y2flat, zflat):
    mesh = plsc.VectorSubcoreMesh(core_axis_name="c", subcore_axis_name="s")

    @functools.partial(
        pl.kernel,
        mesh=mesh,
        out_type=jax.ShapeDtypeStruct((NC, NPAD * N_CLS), jnp.float32),
        scratch_types=[
            pltpu.VMEM((GRP, CH), jnp.int32),
            pltpu.VMEM((GRP, CH), jnp.int32),
            pltpu.VMEM((4, N_CLS, CH), jnp.int32),
            pltpu.VMEM((4, N_CLS, CH), jnp.int32),
            pltpu.VMEM((4, N_CLS, CH), jnp.float32),
            pltpu.VMEM_SHARED((NPAD * N_CLS,), jnp.float32),
            pltpu.SemaphoreType.DMA,
            pltpu.SemaphoreType.DMA,
            pltpu.SemaphoreType.DMA,
            pltpu.SemaphoreType.DMA,
            pltpu.SemaphoreType.DMA,
        ],
    )
    def k(ei_hbm, y_hbm, z_hbm, out_hbm, si_v, di_v, si4_v, di4_v,
          vals_v, acc_sh, sem0, sem1, sem2, sem3, semz):
        c = lax.axis_index("c")
        s = lax.axis_index("s")
        w = _worker(c, s)
        sems = (sem0, sem1, sem2, sem3)
        sl = pl.ds(s * FR, FR)
        wsl = pl.ds(w * GRP, GRP)
        pltpu.async_copy(z_hbm.at[sl], acc_sh.at[sl], semz)
        pltpu.async_copy(ei_hbm.at[0].at[wsl], si_v, semz).wait()
        pltpu.async_copy(ei_hbm.at[1].at[wsl], di_v, semz).wait()
        pltpu.make_async_copy(z_hbm.at[sl], acc_sh.at[sl], semz).wait()
        plsc.subcore_barrier()

        def expand_idx(j, b):
            for t in range(CH // L):
                tsl = pl.ds(t * L, L)
                sv = si_v[j, tsl] * N_CLS
                dv = di_v[j, tsl] * N_CLS
                for cc in range(N_CLS):
                    si4_v[b, cc, tsl] = sv + cc
                    di4_v[b, cc, tsl] = dv + cc

        def fire(b):
            for cc in range(N_CLS):
                pltpu.async_copy(
                    y_hbm.at[si4_v.at[b, cc]], vals_v.at[b, cc], sems[b])

        def drain_scatter(b):
            for cc in range(N_CLS):
                pltpu.make_async_copy(
                    y_hbm.at[si4_v.at[b, cc]], vals_v.at[b, cc],
                    sems[b]).wait()
            for cc in range(N_CLS):
                pltpu.sync_copy(vals_v.at[b, cc], acc_sh.at[di4_v.at[b, cc]],
                                add=True)

        for p in range(3):
            expand_idx(p, p)
            fire(p)

        def body(kk, carry):
            for p in range(4):
                j = 4 * kk + p

                @pl.when(j + 3 < GRP)
                def _():
                    q = (p + 3) % 4
                    expand_idx(j + 3, q)
                    fire(q)

                drain_scatter(p)
            return carry

        lax.fori_loop(0, GRP // 4, body, 0)
        plsc.subcore_barrier()
        pltpu.sync_copy(acc_sh.at[sl], out_hbm.at[c].at[sl])

    return k(ei128, y2flat, zflat)


def _tc_y1(x, w1, degp):
    BLK = 512

    def body(x_ref, w_ref, degp_ref, y_ref):
        deg = degp_ref[0, :] + degp_ref[1, :] + 1.0
        dinv = lax.rsqrt(deg)
        xw = jnp.dot(x_ref[...], w_ref[...], preferred_element_type=jnp.float32)
        y_ref[...] = xw * dinv[:, None]

    return pl.pallas_call(
        body,
        grid=(NPAD // BLK,),
        in_specs=[
            pl.BlockSpec((BLK, D_IN), lambda i: (i, 0)),
            pl.BlockSpec((D_IN, D_HID), lambda i: (0, 0)),
            pl.BlockSpec((NC, BLK), lambda i: (0, i)),
        ],
        out_specs=pl.BlockSpec((BLK, D_HID), lambda i: (i, 0)),
        out_shape=jax.ShapeDtypeStruct((NPAD, D_HID), jnp.float32),
    )(x, w1, degp)


def _tc_y2(aggp, y1, degp, b1, w2):
    BLK = 512

    def body(agg_ref, y1_ref, degp_ref, b1_ref, w2_ref, y2_ref):
        deg = degp_ref[0, :] + degp_ref[1, :] + 1.0
        dinv = lax.rsqrt(deg)
        agg = agg_ref[0] + agg_ref[1] + y1_ref[...]
        h = jnp.maximum(agg * dinv[:, None] + b1_ref[...], 0.0)
        hw = jnp.dot(h, w2_ref[...], preferred_element_type=jnp.float32)
        y2_ref[...] = hw * dinv[:, None]

    return pl.pallas_call(
        body,
        grid=(NPAD // BLK,),
        in_specs=[
            pl.BlockSpec((NC, BLK, D_HID), lambda i: (0, i, 0)),
            pl.BlockSpec((BLK, D_HID), lambda i: (i, 0)),
            pl.BlockSpec((NC, BLK), lambda i: (0, i)),
            pl.BlockSpec((1, D_HID), lambda i: (0, 0)),
            pl.BlockSpec((D_HID, N_CLS), lambda i: (0, 0)),
        ],
        out_specs=pl.BlockSpec((BLK, N_CLS), lambda i: (i, 0)),
        out_shape=jax.ShapeDtypeStruct((NPAD, N_CLS), jnp.float32),
    )(aggp, y1, degp, b1, w2)


def _tc_out(aggp2, y2, degp, b2):
    BLK = 512

    def body(agg_ref, y2_ref, degp_ref, b2_ref, o_ref):
        deg = degp_ref[0, :] + degp_ref[1, :] + 1.0
        dinv = lax.rsqrt(deg)
        agg = agg_ref[0] + agg_ref[1] + y2_ref[...]
        o_ref[...] = agg * dinv[:, None] + b2_ref[...]

    return pl.pallas_call(
        body,
        grid=(NPAD // BLK,),
        in_specs=[
            pl.BlockSpec((NC, BLK, N_CLS), lambda i: (0, i, 0)),
            pl.BlockSpec((BLK, N_CLS), lambda i: (i, 0)),
            pl.BlockSpec((NC, BLK), lambda i: (0, i)),
            pl.BlockSpec((1, N_CLS), lambda i: (0, 0)),
        ],
        out_specs=pl.BlockSpec((BLK, N_CLS), lambda i: (i, 0)),
        out_shape=jax.ShapeDtypeStruct((NPAD, N_CLS), jnp.float32),
    )(aggp2, y2, degp, b2)


def kernel(x, edge_index, W1, b1, W2, b2):
    ei = edge_index.astype(jnp.int32)
    ei_pad = jnp.pad(ei, ((0, 0), (0, E_PAD - E)), constant_values=PAD_IDX)
    ei128 = ei_pad.reshape(2, E_PAD // CH, CH)
    x_p = jnp.pad(x, ((0, NPAD - N), (0, 0)))
    z128 = jnp.zeros((NPAD, D_HID), jnp.float32)

    degp = _sc_degree(ei128)
    y1 = _tc_y1(x_p, W1, degp)
    aggp1 = _sc_agg(ei128, y1, z128, D_HID)
    y2 = _tc_y2(aggp1, y1, degp, b1.reshape(1, D_HID), W2)
    aggp2f = _sc_agg_flat(ei128, y2.reshape(-1),
                          jnp.zeros((NPAD * N_CLS,), jnp.float32))
    aggp2 = aggp2f.reshape(NC, NPAD, N_CLS)
    return _tc_out(aggp2, y2, degp, b2.reshape(1, N_CLS))[:N]


# speedup vs baseline: 1.1091x; 1.1091x over previous
import functools

import jax
import jax.numpy as jnp
from jax import lax
from jax.experimental import pallas as pl
from jax.experimental.pallas import tpu as pltpu
from jax.experimental.pallas import tpu_sc as plsc

N = 10000
E = 320000
D_IN = 128
D_HID = 128
N_CLS = 4

NC = 2
NS = 16
L = 16
NW = NC * NS

CH = 128
GRP = 80
CH1 = 64
G1 = 160
H1 = G1 // 4
E_PAD = NW * GRP * CH
NPAD = 10240
RPW = NPAD // NS
FR = NPAD * N_CLS // NS
PAD_IDX = N


def _worker(c, s):
    return s * NC + c


def _sc_degree(ei128):
    mesh = plsc.VectorSubcoreMesh(core_axis_name="c", subcore_axis_name="s")

    @functools.partial(
        pl.kernel,
        mesh=mesh,
        out_type=jax.ShapeDtypeStruct((NC, NPAD), jnp.float32),
        scratch_types=[
            pltpu.VMEM((GRP, CH), jnp.int32),
            pltpu.VMEM((CH,), jnp.float32),
            pltpu.VMEM((RPW,), jnp.float32),
            pltpu.VMEM_SHARED((NPAD,), jnp.float32),
            pltpu.SemaphoreType.DMA,
        ],
    )
    def k(ei_hbm, out_hbm, idx_v, ones_v, buf_v, acc_sh, sem):
        c = lax.axis_index("c")
        s = lax.axis_index("s")
        w = _worker(c, s)
        for i in range(CH // L):
            ones_v[pl.ds(i * L, L)] = jnp.ones((L,), jnp.float32)

        def zero_body(i, carry):
            buf_v[pl.ds(i * L, L)] = jnp.zeros((L,), jnp.float32)
            return carry

        lax.fori_loop(0, RPW // L, zero_body, 0)
        sl = pl.ds(s * RPW, RPW)
        pltpu.sync_copy(buf_v, acc_sh.at[sl])
        pltpu.async_copy(
            ei_hbm.at[1].at[pl.ds(w * GRP, GRP)], idx_v, sem).wait()
        plsc.subcore_barrier()

        def body(j, carry):
            pltpu.sync_copy(ones_v, acc_sh.at[idx_v.at[j]], add=True)
            return carry

        lax.fori_loop(0, GRP, body, 0)
        plsc.subcore_barrier()
        pltpu.sync_copy(acc_sh.at[sl], out_hbm.at[c].at[sl])

    return k(ei128)


def _sc_agg(ei64, y, zeros, d):
    mesh = plsc.VectorSubcoreMesh(core_axis_name="c", subcore_axis_name="s")

    @functools.partial(
        pl.kernel,
        mesh=mesh,
        out_type=jax.ShapeDtypeStruct((NC, NPAD, d), jnp.float32),
        scratch_types=[
            pltpu.VMEM((H1, CH1), jnp.int32),
            pltpu.VMEM((H1, CH1), jnp.int32),
            pltpu.VMEM((CH1, d), jnp.float32),
            pltpu.VMEM((CH1, d), jnp.float32),
            pltpu.VMEM((CH1, d), jnp.float32),
            pltpu.VMEM((CH1, d), jnp.float32),
            pltpu.VMEM_SHARED((NPAD, d), jnp.float32),
            pltpu.SemaphoreType.DMA,
            pltpu.SemaphoreType.DMA,
            pltpu.SemaphoreType.DMA,
            pltpu.SemaphoreType.DMA,
            pltpu.SemaphoreType.DMA,
        ],
    )
    def k(ei_hbm, y_hbm, z_hbm, out_hbm, si_v, di_v, r0_v, r1_v, r2_v, r3_v,
          acc_sh, sem0, sem1, sem2, sem3, semz):
        c = lax.axis_index("c")
        s = lax.axis_index("s")
        w = _worker(c, s)
        rows = (r0_v, r1_v, r2_v, r3_v)
        sems = (sem0, sem1, sem2, sem3)
        sl = pl.ds(s * RPW, RPW)
        pltpu.async_copy(z_hbm.at[sl], acc_sh.at[sl], semz)
        pltpu.make_async_copy(z_hbm.at[sl], acc_sh.at[sl], semz).wait()
        plsc.subcore_barrier()

        def run_half(h):
            base = w * G1 + h * H1
            pltpu.async_copy(
                ei_hbm.at[0].at[pl.ds(base, H1)], si_v, semz).wait()
            pltpu.async_copy(
                ei_hbm.at[1].at[pl.ds(base, H1)], di_v, semz).wait()
            for p in range(3):
                pltpu.async_copy(y_hbm.at[si_v.at[p]], rows[p], sems[p])

            def body(kk, carry):
                for p in range(4):
                    j = 4 * kk + p
                    pltpu.make_async_copy(
                        y_hbm.at[si_v.at[j]], rows[p], sems[p]).wait()
                    pltpu.sync_copy(rows[p], acc_sh.at[di_v.at[j]], add=True)

                    @pl.when(j + 3 < H1)
                    def _():
                        q = (p + 3) % 4
                        pltpu.async_copy(
                            y_hbm.at[si_v.at[j + 3]], rows[q], sems[q])

                return carry

            lax.fori_loop(0, H1 // 4, body, 0)

        for q in range(4):
            run_half(q)
        plsc.subcore_barrier()
        pltpu.sync_copy(acc_sh.at[sl], out_hbm.at[c].at[sl, :])

    return k(ei64, y, zeros)


def _sc_agg_flat(ei128, y2flat, zflat):
    mesh = plsc.VectorSubcoreMesh(core_axis_name="c", subcore_axis_name="s")

    @functools.partial(
        pl.kernel,
        mesh=mesh,
        out_type=jax.ShapeDtypeStruct((NC, NPAD * N_CLS), jnp.float32),
        scratch_types=[
            pltpu.VMEM((GRP, CH), jnp.int32),
            pltpu.VMEM((GRP, CH), jnp.int32),
            pltpu.VMEM((2, N_CLS, CH), jnp.int32),
            pltpu.VMEM((2, N_CLS, CH), jnp.int32),
            pltpu.VMEM((2, N_CLS, CH), jnp.float32),
            pltpu.VMEM_SHARED((NPAD * N_CLS,), jnp.float32),
            pltpu.SemaphoreType.DMA,
            pltpu.SemaphoreType.DMA,
            pltpu.SemaphoreType.DMA,
        ],
    )
    def k(ei_hbm, y_hbm, z_hbm, out_hbm, si_v, di_v, si4_v, di4_v,
          vals_v, acc_sh, sem0, sem1, semz):
        c = lax.axis_index("c")
        s = lax.axis_index("s")
        w = _worker(c, s)
        sems = (sem0, sem1)
        sl = pl.ds(s * FR, FR)
        wsl = pl.ds(w * GRP, GRP)
        pltpu.async_copy(z_hbm.at[sl], acc_sh.at[sl], semz)
        pltpu.async_copy(ei_hbm.at[0].at[wsl], si_v, semz).wait()
        pltpu.async_copy(ei_hbm.at[1].at[wsl], di_v, semz).wait()
        pltpu.make_async_copy(z_hbm.at[sl], acc_sh.at[sl], semz).wait()
        plsc.subcore_barrier()

        def expand_idx(j, b):
            for t in range(CH // L):
                tsl = pl.ds(t * L, L)
                sv = si_v[j, tsl] * N_CLS
                dv = di_v[j, tsl] * N_CLS
                for cc in range(N_CLS):
                    si4_v[b, cc, tsl] = sv + cc
                    di4_v[b, cc, tsl] = dv + cc

        def fire(b):
            for cc in range(N_CLS):
                pltpu.async_copy(
                    y_hbm.at[si4_v.at[b, cc]], vals_v.at[b, cc], sems[b])

        def drain_scatter(b):
            for cc in range(N_CLS):
                pltpu.make_async_copy(
                    y_hbm.at[si4_v.at[b, cc]], vals_v.at[b, cc],
                    sems[b]).wait()
            for cc in range(N_CLS):
                pltpu.sync_copy(vals_v.at[b, cc], acc_sh.at[di4_v.at[b, cc]],
                                add=True)

        expand_idx(0, 0)
        fire(0)

        def body(kk, carry):
            for p in range(2):
                j = 2 * kk + p

                @pl.when(j + 1 < GRP)
                def _():
                    expand_idx(j + 1, 1 - p)
                    fire(1 - p)

                drain_scatter(p)
            return carry

        lax.fori_loop(0, GRP // 2, body, 0)
        plsc.subcore_barrier()
        pltpu.sync_copy(acc_sh.at[sl], out_hbm.at[c].at[sl])

    return k(ei128, y2flat, zflat)


def _tc_y1(x, w1, degp):
    BLK = 512

    def body(x_ref, w_ref, degp_ref, y_ref):
        deg = degp_ref[0, :] + degp_ref[1, :] + 1.0
        dinv = lax.rsqrt(deg)
        xw = jnp.dot(x_ref[...], w_ref[...], preferred_element_type=jnp.float32)
        y_ref[...] = xw * dinv[:, None]

    return pl.pallas_call(
        body,
        grid=(NPAD // BLK,),
        in_specs=[
            pl.BlockSpec((BLK, D_IN), lambda i: (i, 0)),
            pl.BlockSpec((D_IN, D_HID), lambda i: (0, 0)),
            pl.BlockSpec((NC, BLK), lambda i: (0, i)),
        ],
        out_specs=pl.BlockSpec((BLK, D_HID), lambda i: (i, 0)),
        out_shape=jax.ShapeDtypeStruct((NPAD, D_HID), jnp.float32),
    )(x, w1, degp)


def _tc_y2(aggp, y1, degp, b1, w2):
    BLK = 512

    def body(agg_ref, y1_ref, degp_ref, b1_ref, w2_ref, y2_ref):
        deg = degp_ref[0, :] + degp_ref[1, :] + 1.0
        dinv = lax.rsqrt(deg)
        agg = agg_ref[0] + agg_ref[1] + y1_ref[...]
        h = jnp.maximum(agg * dinv[:, None] + b1_ref[...], 0.0)
        hw = jnp.dot(h, w2_ref[...], preferred_element_type=jnp.float32)
        y2_ref[...] = hw * dinv[:, None]

    return pl.pallas_call(
        body,
        grid=(NPAD // BLK,),
        in_specs=[
            pl.BlockSpec((NC, BLK, D_HID), lambda i: (0, i, 0)),
            pl.BlockSpec((BLK, D_HID), lambda i: (i, 0)),
            pl.BlockSpec((NC, BLK), lambda i: (0, i)),
            pl.BlockSpec((1, D_HID), lambda i: (0, 0)),
            pl.BlockSpec((D_HID, N_CLS), lambda i: (0, 0)),
        ],
        out_specs=pl.BlockSpec((BLK, N_CLS), lambda i: (i, 0)),
        out_shape=jax.ShapeDtypeStruct((NPAD, N_CLS), jnp.float32),
    )(aggp, y1, degp, b1, w2)


def _tc_out(aggp2, y2, degp, b2):
    BLK = 512

    def body(agg_ref, y2_ref, degp_ref, b2_ref, o_ref):
        deg = degp_ref[0, :] + degp_ref[1, :] + 1.0
        dinv = lax.rsqrt(deg)
        agg = agg_ref[0] + agg_ref[1] + y2_ref[...]
        o_ref[...] = agg * dinv[:, None] + b2_ref[...]

    return pl.pallas_call(
        body,
        grid=(NPAD // BLK,),
        in_specs=[
            pl.BlockSpec((NC, BLK, N_CLS), lambda i: (0, i, 0)),
            pl.BlockSpec((BLK, N_CLS), lambda i: (i, 0)),
            pl.BlockSpec((NC, BLK), lambda i: (0, i)),
            pl.BlockSpec((1, N_CLS), lambda i: (0, 0)),
        ],
        out_specs=pl.BlockSpec((BLK, N_CLS), lambda i: (i, 0)),
        out_shape=jax.ShapeDtypeStruct((NPAD, N_CLS), jnp.float32),
    )(aggp2, y2, degp, b2)


def kernel(x, edge_index, W1, b1, W2, b2):
    ei = edge_index.astype(jnp.int32)
    ei_pad = jnp.pad(ei, ((0, 0), (0, E_PAD - E)), constant_values=PAD_IDX)
    ei128 = ei_pad.reshape(2, E_PAD // CH, CH)
    ei64 = ei_pad.reshape(2, E_PAD // CH1, CH1)
    x_p = jnp.pad(x, ((0, NPAD - N), (0, 0)))
    z128 = jnp.zeros((NPAD, D_HID), jnp.float32)

    degp = _sc_degree(ei128)
    y1 = _tc_y1(x_p, W1, degp)
    aggp1 = _sc_agg(ei64, y1, z128, D_HID)
    y2 = _tc_y2(aggp1, y1, degp, b1.reshape(1, D_HID), W2)
    aggp2f = _sc_agg_flat(ei128, y2.reshape(-1),
                          jnp.zeros((NPAD * N_CLS,), jnp.float32))
    aggp2 = aggp2f.reshape(NC, NPAD, N_CLS)
    return _tc_out(aggp2, y2, degp, b2.reshape(1, N_CLS))[:N]
